# baseline (speedup 1.0000x reference)
import functools

import jax
import jax.numpy as jnp
from jax import lax
from jax.experimental import pallas as pl
from jax.experimental.pallas import tpu as pltpu
from jax.experimental.pallas import tpu_sc as plsc

VOCAB = 1000000
DIM = 128
BATCH = 4096

_info = plsc.get_sparse_core_info()
_NC, _NS, _NL = _info.num_cores, _info.num_subcores, _info.num_lanes
_NW = _NC * _NS
_BPW = BATCH // _NW
_CHUNKS = DIM // 32

_mesh = plsc.VectorSubcoreMesh(core_axis_name="c", subcore_axis_name="s")


@functools.partial(
    pl.kernel,
    mesh=_mesh,
    compiler_params=pltpu.CompilerParams(needs_layout_passes=False),
    out_type=jax.ShapeDtypeStruct((BATCH, DIM), jnp.bfloat16),
    scratch_types=[
        pltpu.VMEM((_BPW,), jnp.int32),
        pltpu.VMEM((_BPW, DIM), jnp.float32),
        pltpu.VMEM((_BPW, DIM), jnp.bfloat16),
        pltpu.SemaphoreType.DMA,
    ],
)
def _embed_lookup(table_hbm, idx_hbm, out_hbm, idx_v, rows_v, pk_v, sem):
    wid = lax.axis_index("s") * _NC + lax.axis_index("c")
    base = wid * _BPW
    pltpu.sync_copy(idx_hbm.at[pl.ds(base, _BPW)], idx_v)
    pltpu.async_copy(table_hbm.at[idx_v], rows_v, sem).wait()

    iota = lax.iota(jnp.int32, _NL)
    half = iota % 8
    idx_e = half * 2
    idx_o = half * 2 + 1
    in_lo = iota < 8
    himask = jnp.int32(-65536)

    def _dg(v, ix):
        return v.at[ix].get(mode="promise_in_bounds")

    def _round_hi16(bits):
        lsb = lax.shift_right_logical(bits, 16) & 1
        return bits + 0x7FFF + lsb

    def _round_hi16(bits):
        lsb = lax.shift_right_logical(bits, 16) & 1
        return bits + 0x7FFF + lsb

    def row_body(i, _):
        for j in range(_CHUNKS):
            c0 = plsc.bitcast(rows_v[i, pl.ds(32 * j, _NL)], jnp.int32)
            c1 = plsc.bitcast(rows_v[i, pl.ds(32 * j + _NL, _NL)], jnp.int32)
            ev = lax.select(in_lo, _dg(c0, idx_e), _dg(c1, idx_e))
            od = lax.select(in_lo, _dg(c0, idx_o), _dg(c1, idx_o))
            lo = lax.shift_right_logical(_round_hi16(ev), 16)
            hi = _round_hi16(od) & himask
            pk_v[i, pl.ds(32 * j, 32)] = plsc.bitcast(lo | hi, jnp.bfloat16)
        return 0

    lax.fori_loop(0, _BPW, row_body, 0)
    pltpu.sync_copy(pk_v, out_hbm.at[pl.ds(base, _BPW)])


def kernel(tokens, embed_table):
    B, T = tokens.shape
    idx = tokens.reshape(B).astype(jnp.int32)
    return _embed_lookup(embed_table, idx).reshape(B, T, DIM)


# speedup vs baseline: 1.0798x; 1.0798x over previous
import functools

import jax
import jax.numpy as jnp
from jax import lax
from jax.experimental import pallas as pl
from jax.experimental.pallas import tpu as pltpu
from jax.experimental.pallas import tpu_sc as plsc

VOCAB = 1000000
DIM = 128
BATCH = 4096

_info = plsc.get_sparse_core_info()
_NC, _NS, _NL = _info.num_cores, _info.num_subcores, _info.num_lanes
_NW = _NC * _NS
_BPW = BATCH // _NW
_HALF = _BPW // 2

_mesh = plsc.VectorSubcoreMesh(core_axis_name="c", subcore_axis_name="s")


@functools.partial(
    pl.kernel,
    mesh=_mesh,
    compiler_params=pltpu.CompilerParams(needs_layout_passes=False),
    out_type=jax.ShapeDtypeStruct((BATCH, DIM), jnp.float32),
    scratch_types=[
        pltpu.VMEM((_BPW,), jnp.int32),
        pltpu.VMEM((_BPW, DIM), jnp.float32),
        pltpu.SemaphoreType.DMA,
        pltpu.SemaphoreType.DMA,
        pltpu.SemaphoreType.DMA,
    ],
)
def _gather_rows(table_hbm, idx_hbm, out_hbm, idx_v, rows_v, g0, g1, osem):
    wid = lax.axis_index("s") * _NC + lax.axis_index("c")
    base = wid * _BPW
    pltpu.sync_copy(idx_hbm.at[pl.ds(base, _BPW)], idx_v)
    h0 = pltpu.async_copy(
        table_hbm.at[idx_v.at[pl.ds(0, _HALF)]], rows_v.at[pl.ds(0, _HALF)], g0)
    h1 = pltpu.async_copy(
        table_hbm.at[idx_v.at[pl.ds(_HALF, _HALF)]],
        rows_v.at[pl.ds(_HALF, _HALF)], g1)
    h0.wait()
    o0 = pltpu.async_copy(
        rows_v.at[pl.ds(0, _HALF)], out_hbm.at[pl.ds(base, _HALF)], osem)
    h1.wait()
    o1 = pltpu.async_copy(
        rows_v.at[pl.ds(_HALF, _HALF)],
        out_hbm.at[pl.ds(base + _HALF, _HALF)], osem)
    o0.wait()
    o1.wait()


def kernel(tokens, embed_table):
    B, T = tokens.shape
    idx = tokens.reshape(B).astype(jnp.int32)
    rows = _gather_rows(embed_table, idx)
    return rows.astype(jnp.bfloat16).reshape(B, T, DIM)


# speedup vs baseline: 1.0861x; 1.0058x over previous
import functools

import jax
import jax.numpy as jnp
from jax import lax
from jax.experimental import pallas as pl
from jax.experimental.pallas import tpu as pltpu
from jax.experimental.pallas import tpu_sc as plsc

VOCAB = 1000000
DIM = 128
BATCH = 4096

_info = plsc.get_sparse_core_info()
_NC, _NS, _NL = _info.num_cores, _info.num_subcores, _info.num_lanes
_NW = _NC * _NS
_BPW = BATCH // _NW
_HALF = _BPW // 2

_mesh = plsc.VectorSubcoreMesh(core_axis_name="c", subcore_axis_name="s")


@functools.partial(
    pl.kernel,
    mesh=_mesh,
    compiler_params=pltpu.CompilerParams(
        needs_layout_passes=False,
        skip_device_barrier=True,
        disable_bounds_checks=True,
        disable_semaphore_checks=True,
    ),
    out_type=jax.ShapeDtypeStruct((BATCH, DIM), jnp.float32),
    scratch_types=[
        pltpu.VMEM((_BPW,), jnp.int32),
        pltpu.VMEM((_BPW, DIM), jnp.float32),
        pltpu.SemaphoreType.DMA,
        pltpu.SemaphoreType.DMA,
        pltpu.SemaphoreType.DMA,
    ],
)
def _gather_rows(table_hbm, idx_hbm, out_hbm, idx_v, rows_v, g0, g1, osem):
    wid = lax.axis_index("s") * _NC + lax.axis_index("c")
    base = wid * _BPW
    pltpu.sync_copy(idx_hbm.at[pl.ds(base, _BPW)], idx_v)
    h0 = pltpu.async_copy(
        table_hbm.at[idx_v.at[pl.ds(0, _HALF)]], rows_v.at[pl.ds(0, _HALF)], g0)
    h1 = pltpu.async_copy(
        table_hbm.at[idx_v.at[pl.ds(_HALF, _HALF)]],
        rows_v.at[pl.ds(_HALF, _HALF)], g1)
    h0.wait()
    o0 = pltpu.async_copy(
        rows_v.at[pl.ds(0, _HALF)], out_hbm.at[pl.ds(base, _HALF)], osem)
    h1.wait()
    o1 = pltpu.async_copy(
        rows_v.at[pl.ds(_HALF, _HALF)],
        out_hbm.at[pl.ds(base + _HALF, _HALF)], osem)
    o0.wait()
    o1.wait()


def kernel(tokens, embed_table):
    B, T = tokens.shape
    idx = tokens.reshape(B).astype(jnp.int32)
    rows = _gather_rows(embed_table, idx)
    return rows.astype(jnp.bfloat16).reshape(B, T, DIM)
